# baseline (speedup 1.0000x reference)
import functools

import jax
import jax.numpy as jnp
from jax import lax
from jax.experimental import pallas as pl
from jax.experimental.pallas import tpu as pltpu
from jax.experimental.pallas import tpu_sc as plsc

N = 10000
NP = 10240
E = 320000
BLK = 128
BPT = 80
CH = 4
CB = BPT // CH
NW = 32
EPAD = NW * BPT * BLK
RS = NP // 16

_mesh = plsc.VectorSubcoreMesh(core_axis_name="c", subcore_axis_name="s")


def _make_agg(d):
    @functools.partial(
        pl.kernel,
        mesh=_mesh,
        out_type=jax.ShapeDtypeStruct((2, NP, d), jnp.float32),
        compiler_params=pltpu.CompilerParams(use_tc_tiling_on_sc=False),
        scratch_types=[
            pltpu.VMEM((CB, BLK), jnp.int32),
            pltpu.VMEM((CB, BLK), jnp.int32),
            pltpu.VMEM((BLK, d), jnp.float32),
            pltpu.VMEM((BLK, d), jnp.float32),
            pltpu.VMEM_SHARED((NP, d), jnp.float32),
            pltpu.SemaphoreType.DMA,
            pltpu.SemaphoreType.DMA,
        ],
    )
    def agg(xp, srcs, dsts, zeros, out, src_v, dst_v, buf_a, buf_b, acc,
            sem_a, sem_b):
        c = lax.axis_index("c")
        s = lax.axis_index("s")
        wid = s * 2 + (1 - c)
        pltpu.sync_copy(zeros, acc.at[pl.ds(s * RS, RS)])
        plsc.subcore_barrier()

        def chunk(k, carry):
            pltpu.sync_copy(srcs.at[wid, pl.ds(k * CB, CB)], src_v)
            pltpu.sync_copy(dsts.at[wid, pl.ds(k * CB, CB)], dst_v)
            pltpu.async_copy(xp.at[src_v.at[0]], buf_a, sem_a)

            def pair(p, carry2):
                j0 = 2 * p
                j1 = j0 + 1
                pltpu.async_copy(xp.at[src_v.at[j1]], buf_b, sem_b)
                pltpu.make_async_copy(xp.at[src_v.at[j0]], buf_a,
                                      sem_a).wait()
                pltpu.sync_copy(buf_a, acc.at[dst_v.at[j0]], add=True)

                @pl.when(j0 + 2 < CB)
                def _():
                    pltpu.async_copy(xp.at[src_v.at[j0 + 2]], buf_a, sem_a)

                pltpu.make_async_copy(xp.at[src_v.at[j1]], buf_b,
                                      sem_b).wait()
                pltpu.sync_copy(buf_b, acc.at[dst_v.at[j1]], add=True)
                return carry2

            lax.fori_loop(0, CB // 2, pair, 0)
            return carry

        lax.fori_loop(0, CH, chunk, 0)
        plsc.subcore_barrier()
        pltpu.sync_copy(acc.at[pl.ds(s * RS, RS)],
                        out.at[c, pl.ds(s * RS, RS)])

    return agg


_agg128 = _make_agg(128)
_agg64 = _make_agg(64)


@functools.partial(
    pl.kernel,
    mesh=_mesh,
    out_type=jax.ShapeDtypeStruct((2, NP, 16), jnp.float32),
    scratch_types=[
        pltpu.VMEM((BPT, BLK), jnp.int32),
        pltpu.VMEM((BLK, 16), jnp.float32),
        pltpu.VMEM_SHARED((NP, 16), jnp.float32),
    ],
)
def _deg_kernel(dsts, zeros, ones, out, dst_v, ones_v, acc):
    c = lax.axis_index("c")
    s = lax.axis_index("s")
    wid = s * 2 + c
    pltpu.sync_copy(dsts.at[wid], dst_v)
    pltpu.sync_copy(ones, ones_v)
    pltpu.sync_copy(zeros, acc.at[pl.ds(s * RS, RS)])
    plsc.subcore_barrier()

    def step(j, carry):
        pltpu.sync_copy(ones_v, acc.at[dst_v.at[j]], add=True)
        return carry

    lax.fori_loop(0, BPT, step, 0)
    plsc.subcore_barrier()
    pltpu.sync_copy(acc.at[pl.ds(s * RS, RS)], out.at[c, pl.ds(s * RS, RS)])


_R = 1280


def _tc1_body(x_ref, w_ref, deg_ref, xp_ref):
    deg = deg_ref[0, :, 0:1] + deg_ref[1, :, 0:1] + 1.0
    dis = lax.rsqrt(deg)
    xw = jnp.dot(x_ref[...], w_ref[...], preferred_element_type=jnp.float32)
    xp_ref[...] = xw * dis


def _tc2_body(s1_ref, xp1_ref, deg_ref, b1_ref, w2_ref, x2p_ref):
    deg = deg_ref[0, :, 0:1] + deg_ref[1, :, 0:1] + 1.0
    dis = lax.rsqrt(deg)
    h = jnp.maximum(dis * (s1_ref[0] + s1_ref[1] + xp1_ref[...]) + b1_ref[...],
                    0.0)
    x2p_ref[...] = jnp.dot(h, w2_ref[...],
                           preferred_element_type=jnp.float32) * dis


def _tc3_body(s2_ref, xp2_ref, deg_ref, b2_ref, out_ref):
    deg = deg_ref[0, :, 0:1] + deg_ref[1, :, 0:1] + 1.0
    dis = lax.rsqrt(deg)
    out_ref[...] = jnp.maximum(
        dis * (s2_ref[0] + s2_ref[1] + xp2_ref[...]) + b2_ref[...], 0.0)


def _tc1(xpad, W1, deg2):
    return pl.pallas_call(
        _tc1_body,
        grid=(NP // _R,),
        in_specs=[
            pl.BlockSpec((_R, 128), lambda i: (i, 0)),
            pl.BlockSpec((128, 128), lambda i: (0, 0)),
            pl.BlockSpec((2, _R, 16), lambda i: (0, i, 0)),
        ],
        out_specs=pl.BlockSpec((_R, 128), lambda i: (i, 0)),
        out_shape=jax.ShapeDtypeStruct((NP, 128), jnp.float32),
    )(xpad, W1, deg2)


def _tc2(s1, xp1, deg2, b1, W2):
    return pl.pallas_call(
        _tc2_body,
        grid=(NP // _R,),
        in_specs=[
            pl.BlockSpec((2, _R, 128), lambda i: (0, i, 0)),
            pl.BlockSpec((_R, 128), lambda i: (i, 0)),
            pl.BlockSpec((2, _R, 16), lambda i: (0, i, 0)),
            pl.BlockSpec((1, 128), lambda i: (0, 0)),
            pl.BlockSpec((128, 64), lambda i: (0, 0)),
        ],
        out_specs=pl.BlockSpec((_R, 64), lambda i: (i, 0)),
        out_shape=jax.ShapeDtypeStruct((NP, 64), jnp.float32),
    )(s1, xp1, deg2, b1, W2)


def _tc3(s2, xp2, deg2, b2):
    return pl.pallas_call(
        _tc3_body,
        grid=(NP // _R,),
        in_specs=[
            pl.BlockSpec((2, _R, 64), lambda i: (0, i, 0)),
            pl.BlockSpec((_R, 64), lambda i: (i, 0)),
            pl.BlockSpec((2, _R, 16), lambda i: (0, i, 0)),
            pl.BlockSpec((1, 64), lambda i: (0, 0)),
        ],
        out_specs=pl.BlockSpec((_R, 64), lambda i: (i, 0)),
        out_shape=jax.ShapeDtypeStruct((NP, 64), jnp.float32),
    )(s2, xp2, deg2, b2)


def kernel(x, edge_index, W1, b1, W2, b2):
    src = edge_index[0]
    dst = edge_index[1]
    pad = EPAD - E
    src_p = jnp.concatenate(
        [src, jnp.zeros((pad,), jnp.int32)]).reshape(NW, BPT, BLK)
    pad_dst = N + (jnp.arange(pad, dtype=jnp.int32) % (NP - N))
    dst_p = jnp.concatenate([dst, pad_dst]).reshape(NW, BPT, BLK)
    xpad = jnp.zeros((NP, 128), jnp.float32).at[:N].set(x)

    zeros16 = jnp.zeros((RS, 16), jnp.float32)
    zeros128 = jnp.zeros((RS, 128), jnp.float32)
    zeros64 = jnp.zeros((RS, 64), jnp.float32)
    ones16 = jnp.ones((BLK, 16), jnp.float32)

    deg2 = _deg_kernel(dst_p, zeros16, ones16)
    xp1 = _tc1(xpad, W1, deg2)
    s1 = _agg128(xp1, src_p, dst_p, zeros128)
    xp2 = _tc2(s1, xp1, deg2, b1.reshape(1, 128), W2)
    s2 = _agg64(xp2, src_p, dst_p, zeros64)
    out = _tc3(s2, xp2, deg2, b2.reshape(1, 64))
    return out[:N]


# speedup vs baseline: 3.0023x; 3.0023x over previous
import functools

import jax
import jax.numpy as jnp
from jax import lax
from jax.experimental import pallas as pl
from jax.experimental.pallas import tpu as pltpu
from jax.experimental.pallas import tpu_sc as plsc

N = 10000
NP = 10240
E = 320000
BLK = 128
BPT = 80
CH = 4
CB = BPT // CH
NW = 32
EPAD = NW * BPT * BLK
RS = NP // 16

_mesh = plsc.VectorSubcoreMesh(core_axis_name="c", subcore_axis_name="s")


@functools.partial(
    pl.kernel,
    mesh=_mesh,
    out_type=jax.ShapeDtypeStruct((2, NP, 128), jnp.float32),
    compiler_params=pltpu.CompilerParams(use_tc_tiling_on_sc=False),
    scratch_types=[
        pltpu.VMEM((CB, BLK), jnp.int32),
        pltpu.VMEM((CB, BLK), jnp.int32),
        pltpu.VMEM((BLK, 128), jnp.float32),
        pltpu.VMEM((BLK, 128), jnp.float32),
        pltpu.VMEM_SHARED((NP, 128), jnp.float32),
        pltpu.SemaphoreType.DMA,
        pltpu.SemaphoreType.DMA,
    ],
)
def _agg(xp, srcs, dsts, zeros, out, src_v, dst_v, buf_a, buf_b, acc,
         sem_a, sem_b):
    c = lax.axis_index("c")
    s = lax.axis_index("s")
    wid = s * 2 + c
    pltpu.sync_copy(zeros, acc.at[pl.ds(s * RS, RS)])
    plsc.subcore_barrier()

    def chunk(k, carry):
        pltpu.sync_copy(srcs.at[wid, pl.ds(k * CB, CB)], src_v)
        pltpu.sync_copy(dsts.at[wid, pl.ds(k * CB, CB)], dst_v)
        pltpu.async_copy(xp.at[src_v.at[0]], buf_a, sem_a)

        def pair(p, carry2):
            j0 = 2 * p
            j1 = j0 + 1
            pltpu.async_copy(xp.at[src_v.at[j1]], buf_b, sem_b)
            pltpu.make_async_copy(xp.at[src_v.at[j0]], buf_a, sem_a).wait()
            pltpu.sync_copy(buf_a, acc.at[dst_v.at[j0]], add=True)

            @pl.when(j0 + 2 < CB)
            def _():
                pltpu.async_copy(xp.at[src_v.at[j0 + 2]], buf_a, sem_a)

            pltpu.make_async_copy(xp.at[src_v.at[j1]], buf_b, sem_b).wait()
            pltpu.sync_copy(buf_b, acc.at[dst_v.at[j1]], add=True)
            return carry2

        lax.fori_loop(0, CB // 2, pair, 0)
        return carry

    lax.fori_loop(0, CH, chunk, 0)
    plsc.subcore_barrier()
    pltpu.sync_copy(acc.at[pl.ds(s * RS, RS)],
                    out.at[c, pl.ds(s * RS, RS)])


@functools.partial(
    pl.kernel,
    mesh=_mesh,
    out_type=jax.ShapeDtypeStruct((2, NP // 8, 128), jnp.float32),
    compiler_params=pltpu.CompilerParams(use_tc_tiling_on_sc=False),
    scratch_types=[
        pltpu.VMEM((BPT, BLK), jnp.int32),
        pltpu.VMEM((BLK, 16), jnp.float32),
        pltpu.VMEM((RS, 16), jnp.float32),
        pltpu.VMEM((RS // 8, 128), jnp.float32),
        pltpu.VMEM_SHARED((NP, 16), jnp.float32),
    ],
)
def _deg_kernel(dsts, out, dst_v, ones_v, buf16, buf128, acc):
    c = lax.axis_index("c")
    s = lax.axis_index("s")
    wid = s * 2 + c
    pltpu.sync_copy(dsts.at[wid], dst_v)

    one = jnp.full((16,), 1.0, jnp.float32)
    zero = jnp.zeros((16,), jnp.float32)

    def fill_ones(i, carry):
        ones_v[i, :] = one
        return carry

    lax.fori_loop(0, BLK, fill_ones, 0)

    def fill_zero(i, carry):
        buf16[i, :] = zero
        return carry

    lax.fori_loop(0, RS, fill_zero, 0)
    pltpu.sync_copy(buf16, acc.at[pl.ds(s * RS, RS)])
    plsc.subcore_barrier()

    def step(j, carry):
        pltpu.sync_copy(ones_v, acc.at[dst_v.at[j]], add=True)
        return carry

    lax.fori_loop(0, BPT, step, 0)
    plsc.subcore_barrier()

    pltpu.sync_copy(acc.at[pl.ds(s * RS, RS)], buf16)

    def repack(ro, carry):
        for g in range(8):
            buf128[ro, pl.ds(g * 16, 16)] = buf16[ro * 8 + g, :]
        return carry

    lax.fori_loop(0, RS // 8, repack, 0)
    pltpu.sync_copy(buf128, out.at[c, pl.ds(s * (RS // 8), RS // 8)])


_R = 1280


def _tc1_body(x_ref, w_ref, deg_ref, xp_ref):
    dis = lax.rsqrt(deg_ref[...] + 1.0)
    xw = jnp.dot(x_ref[...], w_ref[...], preferred_element_type=jnp.float32)
    xp_ref[...] = xw * dis


def _tc2_body(s1_ref, xp1_ref, deg_ref, b1_ref, w2_ref, x2p_ref):
    dis = lax.rsqrt(deg_ref[...] + 1.0)
    h = jnp.maximum(dis * (s1_ref[0] + s1_ref[1] + xp1_ref[...]) + b1_ref[...],
                    0.0)
    x2p_ref[...] = jnp.dot(h, w2_ref[...],
                           preferred_element_type=jnp.float32) * dis


def _tc3_body(s2_ref, xp2_ref, deg_ref, b2_ref, out_ref):
    dis = lax.rsqrt(deg_ref[...] + 1.0)
    out_ref[...] = jnp.maximum(
        dis * (s2_ref[0] + s2_ref[1] + xp2_ref[...]) + b2_ref[...], 0.0)


def _tc1(xpad, W1, degc):
    return pl.pallas_call(
        _tc1_body,
        grid=(NP // _R,),
        in_specs=[
            pl.BlockSpec((_R, 128), lambda i: (i, 0)),
            pl.BlockSpec((128, 128), lambda i: (0, 0)),
            pl.BlockSpec((_R, 1), lambda i: (i, 0)),
        ],
        out_specs=pl.BlockSpec((_R, 128), lambda i: (i, 0)),
        out_shape=jax.ShapeDtypeStruct((NP, 128), jnp.float32),
    )(xpad, W1, degc)


def _tc2(s1, xp1, degc, b1, W2p):
    return pl.pallas_call(
        _tc2_body,
        grid=(NP // _R,),
        in_specs=[
            pl.BlockSpec((2, _R, 128), lambda i: (0, i, 0)),
            pl.BlockSpec((_R, 128), lambda i: (i, 0)),
            pl.BlockSpec((_R, 1), lambda i: (i, 0)),
            pl.BlockSpec((1, 128), lambda i: (0, 0)),
            pl.BlockSpec((128, 128), lambda i: (0, 0)),
        ],
        out_specs=pl.BlockSpec((_R, 128), lambda i: (i, 0)),
        out_shape=jax.ShapeDtypeStruct((NP, 128), jnp.float32),
    )(s1, xp1, degc, b1, W2p)


def _tc3(s2, xp2, degc, b2):
    return pl.pallas_call(
        _tc3_body,
        grid=(NP // _R,),
        in_specs=[
            pl.BlockSpec((2, _R, 128), lambda i: (0, i, 0)),
            pl.BlockSpec((_R, 128), lambda i: (i, 0)),
            pl.BlockSpec((_R, 1), lambda i: (i, 0)),
            pl.BlockSpec((1, 128), lambda i: (0, 0)),
        ],
        out_specs=pl.BlockSpec((_R, 128), lambda i: (i, 0)),
        out_shape=jax.ShapeDtypeStruct((NP, 128), jnp.float32),
    )(s2, xp2, degc, b2)


def kernel(x, edge_index, W1, b1, W2, b2):
    src = edge_index[0]
    dst = edge_index[1]
    pad = EPAD - E
    pad_row = N + (jnp.arange(pad, dtype=jnp.int32) % (NP - N))
    src_p = jnp.concatenate([src, pad_row]).reshape(NW, BPT, BLK)
    dst_p = jnp.concatenate([dst, pad_row]).reshape(NW, BPT, BLK)
    xpad = jnp.zeros((NP, 128), jnp.float32).at[:N].set(x)
    zeros128 = jnp.zeros((RS, 128), jnp.float32)

    deg2 = _deg_kernel(dst_p)
    degc = (deg2[0] + deg2[1]).reshape(NP, 16)[:, 0:1]

    W2p = jnp.zeros((128, 128), jnp.float32).at[:, :64].set(W2)
    b1r = b1.reshape(1, 128)
    b2r = jnp.zeros((1, 128), jnp.float32).at[0, :64].set(b2)

    xp1 = _tc1(xpad, W1, degc)
    s1 = _agg(xp1, src_p, dst_p, zeros128)
    xp2 = _tc2(s1, xp1, degc, b1r, W2p)
    s2 = _agg(xp2, src_p, dst_p, zeros128)
    out = _tc3(s2, xp2, degc, b2r)
    return out[:N, :64]


# speedup vs baseline: 3.1105x; 1.0360x over previous
import functools

import jax
import jax.numpy as jnp
from jax import lax
from jax.experimental import pallas as pl
from jax.experimental.pallas import tpu as pltpu
from jax.experimental.pallas import tpu_sc as plsc

N = 10000
NP = 10240
E = 320000
BLK = 128
BPT = 80
CH = 2
CB = BPT // CH
NW = 32
EPAD = NW * BPT * BLK
RS = NP // 16

_mesh = plsc.VectorSubcoreMesh(core_axis_name="c", subcore_axis_name="s")


@functools.partial(
    pl.kernel,
    mesh=_mesh,
    out_type=jax.ShapeDtypeStruct((2, NP, 128), jnp.float32),
    compiler_params=pltpu.CompilerParams(use_tc_tiling_on_sc=False),
    scratch_types=[
        pltpu.VMEM((CB, BLK), jnp.int32),
        pltpu.VMEM((CB, BLK), jnp.int32),
        pltpu.VMEM((BLK, 128), jnp.float32),
        pltpu.VMEM((BLK, 128), jnp.float32),
        pltpu.VMEM_SHARED((NP, 128), jnp.float32),
        pltpu.SemaphoreType.DMA,
        pltpu.SemaphoreType.DMA,
    ],
)
def _agg(xp, srcs, dsts, zeros, out, src_v, dst_v, buf_a, buf_b, acc,
         sem_a, sem_b):
    c = lax.axis_index("c")
    s = lax.axis_index("s")
    wid = s * 2 + c
    pltpu.sync_copy(zeros, acc.at[pl.ds(s * RS, RS)])
    plsc.subcore_barrier()

    def chunk(k, carry):
        pltpu.sync_copy(srcs.at[wid, pl.ds(k * CB, CB)], src_v)
        pltpu.sync_copy(dsts.at[wid, pl.ds(k * CB, CB)], dst_v)
        pltpu.async_copy(xp.at[src_v.at[0]], buf_a, sem_a)

        def pair(p, carry2):
            j0 = 2 * p
            j1 = j0 + 1
            pltpu.async_copy(xp.at[src_v.at[j1]], buf_b, sem_b)
            pltpu.make_async_copy(xp.at[src_v.at[j0]], buf_a, sem_a).wait()
            pltpu.sync_copy(buf_a, acc.at[dst_v.at[j0]], add=True)

            @pl.when(j0 + 2 < CB)
            def _():
                pltpu.async_copy(xp.at[src_v.at[j0 + 2]], buf_a, sem_a)

            pltpu.make_async_copy(xp.at[src_v.at[j1]], buf_b, sem_b).wait()
            pltpu.sync_copy(buf_b, acc.at[dst_v.at[j1]], add=True)
            return carry2

        lax.fori_loop(0, CB // 2, pair, 0)
        return carry

    lax.fori_loop(0, CH, chunk, 0)
    plsc.subcore_barrier()
    pltpu.sync_copy(acc.at[pl.ds(s * RS, RS)],
                    out.at[c, pl.ds(s * RS, RS)])


@functools.partial(
    pl.kernel,
    mesh=_mesh,
    out_type=jax.ShapeDtypeStruct((2, NP // 8, 128), jnp.float32),
    compiler_params=pltpu.CompilerParams(use_tc_tiling_on_sc=False),
    scratch_types=[
        pltpu.VMEM((BPT, BLK), jnp.int32),
        pltpu.VMEM((BLK, 16), jnp.float32),
        pltpu.VMEM((RS, 16), jnp.float32),
        pltpu.VMEM((RS // 8, 128), jnp.float32),
        pltpu.VMEM_SHARED((NP, 16), jnp.float32),
    ],
)
def _deg_kernel(dsts, out, dst_v, ones_v, buf16, buf128, acc):
    c = lax.axis_index("c")
    s = lax.axis_index("s")
    wid = s * 2 + c
    pltpu.sync_copy(dsts.at[wid], dst_v)

    one = jnp.full((16,), 1.0, jnp.float32)
    zero = jnp.zeros((16,), jnp.float32)

    def fill_ones(i, carry):
        ones_v[i, :] = one
        return carry

    lax.fori_loop(0, BLK, fill_ones, 0)

    def fill_zero(i, carry):
        buf16[i, :] = zero
        return carry

    lax.fori_loop(0, RS, fill_zero, 0)
    pltpu.sync_copy(buf16, acc.at[pl.ds(s * RS, RS)])
    plsc.subcore_barrier()

    def step(j, carry):
        pltpu.sync_copy(ones_v, acc.at[dst_v.at[j]], add=True)
        return carry

    lax.fori_loop(0, BPT, step, 0)
    plsc.subcore_barrier()

    pltpu.sync_copy(acc.at[pl.ds(s * RS, RS)], buf16)

    def repack(ro, carry):
        for g in range(8):
            buf128[ro, pl.ds(g * 16, 16)] = buf16[ro * 8 + g, :]
        return carry

    lax.fori_loop(0, RS // 8, repack, 0)
    pltpu.sync_copy(buf128, out.at[c, pl.ds(s * (RS // 8), RS // 8)])


_R = 1280


def _tc1_body(x_ref, w_ref, deg_ref, xp_ref):
    dis = lax.rsqrt(deg_ref[...] + 1.0)
    xw = jnp.dot(x_ref[...], w_ref[...], preferred_element_type=jnp.float32)
    xp_ref[...] = xw * dis


def _tc2_body(s1_ref, xp1_ref, deg_ref, b1_ref, w2_ref, x2p_ref):
    dis = lax.rsqrt(deg_ref[...] + 1.0)
    h = jnp.maximum(dis * (s1_ref[0] + s1_ref[1] + xp1_ref[...]) + b1_ref[...],
                    0.0)
    x2p_ref[...] = jnp.dot(h, w2_ref[...],
                           preferred_element_type=jnp.float32) * dis


def _tc3_body(s2_ref, xp2_ref, deg_ref, b2_ref, out_ref):
    dis = lax.rsqrt(deg_ref[...] + 1.0)
    out_ref[...] = jnp.maximum(
        dis * (s2_ref[0] + s2_ref[1] + xp2_ref[...]) + b2_ref[...], 0.0)


def _tc1(xpad, W1, degc):
    return pl.pallas_call(
        _tc1_body,
        grid=(NP // _R,),
        in_specs=[
            pl.BlockSpec((_R, 128), lambda i: (i, 0)),
            pl.BlockSpec((128, 128), lambda i: (0, 0)),
            pl.BlockSpec((_R, 1), lambda i: (i, 0)),
        ],
        out_specs=pl.BlockSpec((_R, 128), lambda i: (i, 0)),
        out_shape=jax.ShapeDtypeStruct((NP, 128), jnp.float32),
    )(xpad, W1, degc)


def _tc2(s1, xp1, degc, b1, W2p):
    return pl.pallas_call(
        _tc2_body,
        grid=(NP // _R,),
        in_specs=[
            pl.BlockSpec((2, _R, 128), lambda i: (0, i, 0)),
            pl.BlockSpec((_R, 128), lambda i: (i, 0)),
            pl.BlockSpec((_R, 1), lambda i: (i, 0)),
            pl.BlockSpec((1, 128), lambda i: (0, 0)),
            pl.BlockSpec((128, 128), lambda i: (0, 0)),
        ],
        out_specs=pl.BlockSpec((_R, 128), lambda i: (i, 0)),
        out_shape=jax.ShapeDtypeStruct((NP, 128), jnp.float32),
    )(s1, xp1, degc, b1, W2p)


def _tc3(s2, xp2, degc, b2):
    return pl.pallas_call(
        _tc3_body,
        grid=(NP // _R,),
        in_specs=[
            pl.BlockSpec((2, _R, 128), lambda i: (0, i, 0)),
            pl.BlockSpec((_R, 128), lambda i: (i, 0)),
            pl.BlockSpec((_R, 1), lambda i: (i, 0)),
            pl.BlockSpec((1, 128), lambda i: (0, 0)),
        ],
        out_specs=pl.BlockSpec((_R, 128), lambda i: (i, 0)),
        out_shape=jax.ShapeDtypeStruct((NP, 128), jnp.float32),
    )(s2, xp2, degc, b2)


def kernel(x, edge_index, W1, b1, W2, b2):
    src = edge_index[0]
    dst = edge_index[1]
    pad = EPAD - E
    pad_row = jnp.broadcast_to(
        N + jnp.arange(NP - N, dtype=jnp.int32),
        (pad // (NP - N), NP - N)).reshape(pad)
    src_p = jnp.concatenate([src, pad_row]).reshape(NW, BPT, BLK)
    dst_p = jnp.concatenate([dst, pad_row]).reshape(NW, BPT, BLK)
    xpad = jnp.zeros((NP, 128), jnp.float32).at[:N].set(x)
    zeros128 = jnp.zeros((RS, 128), jnp.float32)

    deg2 = _deg_kernel(dst_p)
    degc = (deg2[0] + deg2[1]).reshape(NP, 16)[:, 0:1]

    W2p = jnp.zeros((128, 128), jnp.float32).at[:, :64].set(W2)
    b1r = b1.reshape(1, 128)
    b2r = jnp.zeros((1, 128), jnp.float32).at[0, :64].set(b2)

    xp1 = _tc1(xpad, W1, degc)
    s1 = _agg(xp1, src_p, dst_p, zeros128)
    xp2 = _tc2(s1, xp1, degc, b1r, W2p)
    s2 = _agg(xp2, src_p, dst_p, zeros128)
    out = _tc3(s2, xp2, degc, b2r)
    return out[:N, :64]


# speedup vs baseline: 3.1187x; 1.0026x over previous
import functools

import jax
import jax.numpy as jnp
from jax import lax
from jax.experimental import pallas as pl
from jax.experimental.pallas import tpu as pltpu
from jax.experimental.pallas import tpu_sc as plsc

N = 10000
NP = 10240
E = 320000
BLK = 128
BPT = 80
CH = 2
CB = BPT // CH
NW = 32
EPAD = NW * BPT * BLK
RS = NP // 16

_mesh = plsc.VectorSubcoreMesh(core_axis_name="c", subcore_axis_name="s")


@functools.partial(
    pl.kernel,
    mesh=_mesh,
    out_type=jax.ShapeDtypeStruct((2, NP, 128), jnp.float32),
    compiler_params=pltpu.CompilerParams(use_tc_tiling_on_sc=False),
    scratch_types=[
        pltpu.VMEM((CB, BLK), jnp.int32),
        pltpu.VMEM((CB, BLK), jnp.int32),
        pltpu.VMEM((BLK, 128), jnp.float32),
        pltpu.VMEM((BLK, 128), jnp.float32),
        pltpu.VMEM_SHARED((NP, 128), jnp.float32),
        pltpu.SemaphoreType.DMA,
        pltpu.SemaphoreType.DMA,
    ],
)
def _agg(xp, srcs, dsts, zeros, out, src_v, dst_v, buf_a, buf_b, acc,
         sem_a, sem_b):
    c = lax.axis_index("c")
    s = lax.axis_index("s")
    wid = s * 2 + c
    pltpu.sync_copy(zeros, acc.at[pl.ds(s * RS, RS)])
    plsc.subcore_barrier()

    def chunk(k, carry):
        pltpu.sync_copy(srcs.at[wid, pl.ds(k * CB, CB)], src_v)
        pltpu.sync_copy(dsts.at[wid, pl.ds(k * CB, CB)], dst_v)
        pltpu.async_copy(xp.at[src_v.at[0]], buf_a, sem_a)

        def pair(p, carry2):
            j0 = 2 * p
            j1 = j0 + 1
            pltpu.async_copy(xp.at[src_v.at[j1]], buf_b, sem_b)
            pltpu.make_async_copy(xp.at[src_v.at[j0]], buf_a, sem_a).wait()
            pltpu.sync_copy(buf_a, acc.at[dst_v.at[j0]], add=True)

            @pl.when(j0 + 2 < CB)
            def _():
                pltpu.async_copy(xp.at[src_v.at[j0 + 2]], buf_a, sem_a)

            pltpu.make_async_copy(xp.at[src_v.at[j1]], buf_b, sem_b).wait()
            pltpu.sync_copy(buf_b, acc.at[dst_v.at[j1]], add=True)
            return carry2

        lax.fori_loop(0, CB // 2, pair, 0)
        return carry

    lax.fori_loop(0, CH, chunk, 0)
    plsc.subcore_barrier()
    pltpu.sync_copy(acc.at[pl.ds(s * RS, RS)],
                    out.at[c, pl.ds(s * RS, RS)])


@functools.partial(
    pl.kernel,
    mesh=_mesh,
    out_type=jax.ShapeDtypeStruct((2, NP // 8, 128), jnp.float32),
    compiler_params=pltpu.CompilerParams(use_tc_tiling_on_sc=False),
    scratch_types=[
        pltpu.VMEM((BPT, BLK), jnp.int32),
        pltpu.VMEM((BLK, 16), jnp.float32),
        pltpu.VMEM((RS, 16), jnp.float32),
        pltpu.VMEM((RS // 8, 128), jnp.float32),
        pltpu.VMEM_SHARED((NP, 16), jnp.float32),
    ],
)
def _deg_kernel(dsts, out, dst_v, ones_v, buf16, buf128, acc):
    c = lax.axis_index("c")
    s = lax.axis_index("s")
    wid = s * 2 + c
    pltpu.sync_copy(dsts.at[wid], dst_v)

    one = jnp.full((16,), 1.0, jnp.float32)
    zero = jnp.zeros((16,), jnp.float32)

    def fill_ones(i, carry):
        ones_v[i, :] = one
        return carry

    lax.fori_loop(0, BLK, fill_ones, 0)

    def fill_zero(i, carry):
        buf16[i, :] = zero
        return carry

    lax.fori_loop(0, RS, fill_zero, 0)
    pltpu.sync_copy(buf16, acc.at[pl.ds(s * RS, RS)])
    plsc.subcore_barrier()

    def step(j, carry):
        pltpu.sync_copy(ones_v, acc.at[dst_v.at[j]], add=True)
        return carry

    lax.fori_loop(0, BPT, step, 0)
    plsc.subcore_barrier()

    pltpu.sync_copy(acc.at[pl.ds(s * RS, RS)], buf16)

    def repack(ro, carry):
        for g in range(8):
            buf128[ro, pl.ds(g * 16, 16)] = buf16[ro * 8 + g, :]
        return carry

    lax.fori_loop(0, RS // 8, repack, 0)
    pltpu.sync_copy(buf128, out.at[c, pl.ds(s * (RS // 8), RS // 8)])


_R = 1280


def _mm_body(x_ref, w_ref, xw_ref):
    xw_ref[...] = jnp.dot(x_ref[...], w_ref[...],
                          preferred_element_type=jnp.float32)


def _scale_body(xw_ref, deg_ref, xp_ref):
    dis = lax.rsqrt(deg_ref[...] + 1.0)
    xp_ref[...] = xw_ref[...] * dis


def _tc2_body(s1_ref, xp1_ref, deg_ref, b1_ref, w2_ref, x2p_ref):
    dis = lax.rsqrt(deg_ref[...] + 1.0)
    h = jnp.maximum(dis * (s1_ref[0] + s1_ref[1] + xp1_ref[...]) + b1_ref[...],
                    0.0)
    x2p_ref[...] = jnp.dot(h, w2_ref[...],
                           preferred_element_type=jnp.float32) * dis


def _tc3_body(s2_ref, xp2_ref, deg_ref, b2_ref, out_ref):
    dis = lax.rsqrt(deg_ref[...] + 1.0)
    out_ref[...] = jnp.maximum(
        dis * (s2_ref[0] + s2_ref[1] + xp2_ref[...]) + b2_ref[...], 0.0)


def _mm(xpad, W1):
    return pl.pallas_call(
        _mm_body,
        grid=(NP // _R,),
        in_specs=[
            pl.BlockSpec((_R, 128), lambda i: (i, 0)),
            pl.BlockSpec((128, 128), lambda i: (0, 0)),
        ],
        out_specs=pl.BlockSpec((_R, 128), lambda i: (i, 0)),
        out_shape=jax.ShapeDtypeStruct((NP, 128), jnp.float32),
    )(xpad, W1)


def _scale(xw, degc):
    return pl.pallas_call(
        _scale_body,
        grid=(NP // _R,),
        in_specs=[
            pl.BlockSpec((_R, 128), lambda i: (i, 0)),
            pl.BlockSpec((_R, 1), lambda i: (i, 0)),
        ],
        out_specs=pl.BlockSpec((_R, 128), lambda i: (i, 0)),
        out_shape=jax.ShapeDtypeStruct((NP, 128), jnp.float32),
    )(xw, degc)


def _tc2(s1, xp1, degc, b1, W2p):
    return pl.pallas_call(
        _tc2_body,
        grid=(NP // _R,),
        in_specs=[
            pl.BlockSpec((2, _R, 128), lambda i: (0, i, 0)),
            pl.BlockSpec((_R, 128), lambda i: (i, 0)),
            pl.BlockSpec((_R, 1), lambda i: (i, 0)),
            pl.BlockSpec((1, 128), lambda i: (0, 0)),
            pl.BlockSpec((128, 128), lambda i: (0, 0)),
        ],
        out_specs=pl.BlockSpec((_R, 128), lambda i: (i, 0)),
        out_shape=jax.ShapeDtypeStruct((NP, 128), jnp.float32),
    )(s1, xp1, degc, b1, W2p)


def _tc3(s2, xp2, degc, b2):
    return pl.pallas_call(
        _tc3_body,
        grid=(NP // _R,),
        in_specs=[
            pl.BlockSpec((2, _R, 128), lambda i: (0, i, 0)),
            pl.BlockSpec((_R, 128), lambda i: (i, 0)),
            pl.BlockSpec((_R, 1), lambda i: (i, 0)),
            pl.BlockSpec((1, 128), lambda i: (0, 0)),
        ],
        out_specs=pl.BlockSpec((_R, 128), lambda i: (i, 0)),
        out_shape=jax.ShapeDtypeStruct((NP, 128), jnp.float32),
    )(s2, xp2, degc, b2)


def kernel(x, edge_index, W1, b1, W2, b2):
    src = edge_index[0]
    dst = edge_index[1]
    pad = EPAD - E
    pad_row = jnp.broadcast_to(
        N + jnp.arange(NP - N, dtype=jnp.int32),
        (pad // (NP - N), NP - N)).reshape(pad)
    src_p = jnp.concatenate([src, pad_row]).reshape(NW, BPT, BLK)
    dst_p = jnp.concatenate([dst, pad_row]).reshape(NW, BPT, BLK)
    xpad = jnp.zeros((NP, 128), jnp.float32).at[:N].set(x)
    zeros128 = jnp.zeros((RS, 128), jnp.float32)

    deg2 = _deg_kernel(dst_p)
    degc = (deg2[0] + deg2[1]).reshape(NP, 16)[:, 0:1]

    W2p = jnp.zeros((128, 128), jnp.float32).at[:, :64].set(W2)
    b1r = b1.reshape(1, 128)
    b2r = jnp.zeros((1, 128), jnp.float32).at[0, :64].set(b2)

    xw1 = _mm(xpad, W1)
    xp1 = _scale(xw1, degc)
    s1 = _agg(xp1, src_p, dst_p, zeros128)
    xp2 = _tc2(s1, xp1, degc, b1r, W2p)
    s2 = _agg(xp2, src_p, dst_p, zeros128)
    out = _tc3(s2, xp2, degc, b2r)
    return out[:N, :64]
